# Initial kernel scaffold; baseline (speedup 1.0000x reference)
import functools

import jax
import jax.numpy as jnp
from jax import lax
from jax.experimental import pallas as pl
from jax.experimental.pallas import tpu as pltpu
from jax.experimental.pallas import tpu_sc as plsc

N_NODES = 10000
N_EDGES = 320000
D = 128

NC = 2
NS = 16
NW = NC * NS
EPT = N_EDGES // NW
CH = 80
NSTEP = EPT // CH
RPT = N_NODES // NS


def _sc_body(compute_deg, x_hbm, src_hbm, dst_hbm, *refs):
    if compute_deg:
        acc_hbm, deg_hbm, acc_sh, zbuf, rows, sidx, didx, sem, dall, degp = refs
    else:
        acc_hbm, acc_sh, zbuf, rows, sidx, didx, sem = refs

    c = lax.axis_index("c")
    s = lax.axis_index("s")
    wid = s * NC + c
    ebase = wid * EPT

    @pl.loop(0, 128)
    def _zero_zbuf(i):
        for j in range(D // 16):
            zbuf[i, pl.ds(j * 16, 16)] = jnp.zeros((16,), jnp.float32)

    @pl.loop(0, RPT // 125)
    def _zero_acc(k):
        pltpu.sync_copy(zbuf.at[pl.ds(0, 125)],
                        acc_sh.at[pl.ds(s * RPT + k * 125, 125)])

    if compute_deg:
        @pl.loop(0, N_NODES // 16)
        def _zero_deg(i):
            degp[pl.ds(i * 16, 16)] = jnp.zeros((16,), jnp.float32)

        pltpu.sync_copy(dst_hbm.at[pl.ds(ebase, EPT)], dall)
        ones = jnp.ones((16,), jnp.float32)

        @pl.loop(0, EPT // 16)
        def _deg(i):
            idx = dall[pl.ds(i * 16, 16)]
            plsc.addupdate_scatter(degp, [idx], ones)

        pltpu.sync_copy(degp, deg_hbm.at[wid])

    plsc.subcore_barrier()

    @pl.loop(0, NSTEP)
    def _edges(i):
        pltpu.sync_copy(src_hbm.at[pl.ds(ebase + i * CH, CH)], sidx)
        pltpu.sync_copy(dst_hbm.at[pl.ds(ebase + i * CH, CH)], didx)
        pltpu.async_copy(x_hbm.at[sidx], rows, sem).wait()
        pltpu.sync_copy(rows, acc_sh.at[didx], add=True)

    plsc.subcore_barrier()

    pltpu.sync_copy(acc_sh.at[pl.ds(s * RPT, RPT)],
                    acc_hbm.at[pl.ds((c * NS + s) * RPT, RPT)])


def _make_sc(compute_deg):
    mesh = plsc.VectorSubcoreMesh(core_axis_name="c", subcore_axis_name="s",
                                  num_cores=NC, num_subcores=NS)
    out_type = [jax.ShapeDtypeStruct((NC * N_NODES, D), jnp.float32)]
    if compute_deg:
        out_type.append(jax.ShapeDtypeStruct((NW, N_NODES), jnp.float32))
    scratch = [
        pltpu.VMEM_SHARED((N_NODES, D), jnp.float32),
        pltpu.VMEM((128, D), jnp.float32),
        pltpu.VMEM((CH, D), jnp.float32),
        pltpu.VMEM((CH,), jnp.int32),
        pltpu.VMEM((CH,), jnp.int32),
        pltpu.SemaphoreType.DMA,
    ]
    if compute_deg:
        scratch += [
            pltpu.VMEM((EPT,), jnp.int32),
            pltpu.VMEM((N_NODES,), jnp.float32),
        ]
    return pl.kernel(functools.partial(_sc_body, compute_deg),
                     out_type=tuple(out_type), mesh=mesh,
                     scratch_types=tuple(scratch))


_sc_agg_deg = _make_sc(True)
_sc_agg = _make_sc(False)


ROWS_BLK = 2000
GRID = (N_NODES // ROWS_BLK,)


def _mean(acc_ref, deg_ref):
    deg = jnp.sum(deg_ref[...], axis=0)
    inv = 1.0 / jnp.maximum(deg, 1.0)
    return (acc_ref[0] + acc_ref[1]) * inv[:, None]


def _tc1_body(acc_ref, deg_ref, x_ref, wl_ref, bl_ref, wr_ref, o_ref):
    mean = _mean(acc_ref, deg_ref)
    h = (jnp.dot(mean, wl_ref[...], preferred_element_type=jnp.float32)
         + bl_ref[...]
         + jnp.dot(x_ref[...], wr_ref[...], preferred_element_type=jnp.float32))
    o_ref[...] = jnp.maximum(h, 0.0)


def _tc2_body(acc_ref, deg_ref, h_ref, wl_ref, bl_ref, wr_ref, wc_ref, bc_ref,
              o_ref):
    mean = _mean(acc_ref, deg_ref)
    h = (jnp.dot(mean, wl_ref[...], preferred_element_type=jnp.float32)
         + bl_ref[...]
         + jnp.dot(h_ref[...], wr_ref[...], preferred_element_type=jnp.float32))
    h = jnp.maximum(h, 0.0)
    z = jnp.dot(h, wc_ref[...], preferred_element_type=jnp.float32) + bc_ref[...]
    m = jnp.max(z, axis=1, keepdims=True)
    e = jnp.exp(z - m)
    o_ref[...] = (z - m) - jnp.log(jnp.sum(e, axis=1, keepdims=True))


_acc_spec = pl.BlockSpec((2, ROWS_BLK, D), lambda i: (0, i, 0))
_deg_spec = pl.BlockSpec((NW, ROWS_BLK), lambda i: (0, i))
_row_spec = pl.BlockSpec((ROWS_BLK, D), lambda i: (i, 0))
_w_spec = pl.BlockSpec((D, D), lambda i: (0, 0))
_b_spec = pl.BlockSpec((1, D), lambda i: (0, 0))

_tc1 = pl.pallas_call(
    _tc1_body, grid=GRID,
    in_specs=[_acc_spec, _deg_spec, _row_spec, _w_spec, _b_spec, _w_spec],
    out_specs=_row_spec,
    out_shape=jax.ShapeDtypeStruct((N_NODES, D), jnp.float32))

_tc2 = pl.pallas_call(
    _tc2_body, grid=GRID,
    in_specs=[_acc_spec, _deg_spec, _row_spec, _w_spec, _b_spec, _w_spec,
              _w_spec, _b_spec],
    out_specs=_row_spec,
    out_shape=jax.ShapeDtypeStruct((N_NODES, D), jnp.float32))


@jax.jit
def kernel(x, edge_index, W1l, b1l, W1r, W2l, b2l, W2r, Wc, bc):
    ei = edge_index.astype(jnp.int32)
    src = ei[0]
    dst = ei[1]
    acc1, degp = _sc_agg_deg(x, src, dst)
    h1 = _tc1(acc1.reshape(2, N_NODES, D), degp, x,
              W1l.T, b1l.reshape(1, D), W1r.T)
    acc2, = _sc_agg(h1, src, dst)
    out = _tc2(acc2.reshape(2, N_NODES, D), degp, h1,
               W2l.T, b2l.reshape(1, D), W2r.T, Wc.T, bc.reshape(1, D))
    return out


# speedup vs baseline: 5.7233x; 5.7233x over previous
import functools

import jax
import jax.numpy as jnp
from jax import lax
from jax.experimental import pallas as pl
from jax.experimental.pallas import tpu as pltpu
from jax.experimental.pallas import tpu_sc as plsc

N_NODES = 10000
N_EDGES = 320000
D = 128

NC = 2
NS = 16
NW = NC * NS
EPT = N_EDGES // NW
CH = 80
NSTEP = EPT // CH
RPT = 624


def _sc_body(compute_deg, x_hbm, src_hbm, dst_hbm, *refs):
    if compute_deg:
        acc_hbm, deg_hbm, acc_sh, zbuf, rows, sidx, didx, sem, dall, degp = refs
    else:
        acc_hbm, acc_sh, zbuf, rows, sidx, didx, sem = refs

    c = lax.axis_index("c")
    s = lax.axis_index("s")
    wid = s * NC + c
    ebase = wid * EPT

    @pl.loop(0, 128)
    def _zero_zbuf(i):
        for j in range(D // 16):
            zbuf[i, pl.ds(j * 16, 16)] = jnp.zeros((16,), jnp.float32)

    rbase = s * RPT

    @pl.loop(0, 4)
    def _zero_acc(k):
        pltpu.sync_copy(zbuf, acc_sh.at[pl.ds(rbase + k * 128, 128)])

    pltpu.sync_copy(zbuf.at[pl.ds(0, RPT - 512)],
                    acc_sh.at[pl.ds(rbase + 512, RPT - 512)])

    @pl.when(s == NS - 1)
    def _zero_tail():
        pltpu.sync_copy(zbuf.at[pl.ds(0, N_NODES - NS * RPT)],
                        acc_sh.at[pl.ds(NS * RPT, N_NODES - NS * RPT)])

    if compute_deg:
        @pl.loop(0, N_NODES // 16)
        def _zero_deg(i):
            degp[pl.ds(i * 16, 16)] = jnp.zeros((16,), jnp.float32)

        pltpu.sync_copy(dst_hbm.at[pl.ds(ebase, EPT)], dall)
        ones = jnp.ones((16,), jnp.float32)

        @pl.loop(0, EPT // 16)
        def _deg(i):
            idx = dall[pl.ds(i * 16, 16)]
            plsc.addupdate_scatter(degp, [idx], ones)

        pltpu.sync_copy(degp, deg_hbm.at[pl.ds(wid * N_NODES, N_NODES)])

    plsc.subcore_barrier()

    @pl.loop(0, NSTEP)
    def _edges(i):
        pltpu.sync_copy(src_hbm.at[pl.ds(ebase + i * CH, CH)], sidx)
        pltpu.sync_copy(dst_hbm.at[pl.ds(ebase + i * CH, CH)], didx)
        pltpu.async_copy(x_hbm.at[sidx], rows, sem).wait()
        pltpu.sync_copy(rows, acc_sh.at[didx], add=True)

    plsc.subcore_barrier()

    pltpu.sync_copy(acc_sh.at[pl.ds(rbase, RPT)],
                    acc_hbm.at[pl.ds(c * N_NODES + rbase, RPT)])

    @pl.when(s == NS - 1)
    def _write_tail():
        pltpu.sync_copy(acc_sh.at[pl.ds(NS * RPT, N_NODES - NS * RPT)],
                        acc_hbm.at[pl.ds(c * N_NODES + NS * RPT,
                                         N_NODES - NS * RPT)])


@functools.lru_cache(maxsize=None)
def _make_sc(compute_deg):
    mesh = plsc.VectorSubcoreMesh(core_axis_name="c", subcore_axis_name="s",
                                  num_cores=NC, num_subcores=NS)
    out_type = [jax.ShapeDtypeStruct((NC * N_NODES, D), jnp.float32)]
    if compute_deg:
        out_type.append(jax.ShapeDtypeStruct((NW * N_NODES,), jnp.float32))
    scratch = [
        pltpu.VMEM_SHARED((N_NODES, D), jnp.float32),
        pltpu.VMEM((128, D), jnp.float32),
        pltpu.VMEM((CH, D), jnp.float32),
        pltpu.VMEM((CH,), jnp.int32),
        pltpu.VMEM((CH,), jnp.int32),
        pltpu.SemaphoreType.DMA,
    ]
    if compute_deg:
        scratch += [
            pltpu.VMEM((EPT,), jnp.int32),
            pltpu.VMEM((N_NODES,), jnp.float32),
        ]
    return pl.kernel(functools.partial(_sc_body, compute_deg),
                     out_type=tuple(out_type), mesh=mesh,
                     scratch_types=tuple(scratch),
                     compiler_params=pltpu.CompilerParams(
                         needs_layout_passes=False))


ROWS_BLK = N_NODES
GRID = (1,)


def _mean(acc_ref, deg_ref):
    deg = jnp.sum(deg_ref[...], axis=0)
    inv = 1.0 / jnp.maximum(deg, 1.0)
    return (acc_ref[0] + acc_ref[1]) * inv[:, None]


def _tc1_body(acc_ref, deg_ref, x_ref, wl_ref, bl_ref, wr_ref, o_ref):
    mean = _mean(acc_ref, deg_ref)
    h = (jnp.dot(mean, wl_ref[...], preferred_element_type=jnp.float32)
         + bl_ref[...]
         + jnp.dot(x_ref[...], wr_ref[...], preferred_element_type=jnp.float32))
    o_ref[...] = jnp.maximum(h, 0.0)


def _tc2_body(acc_ref, deg_ref, h_ref, wl_ref, bl_ref, wr_ref, wc_ref, bc_ref,
              o_ref):
    mean = _mean(acc_ref, deg_ref)
    h = (jnp.dot(mean, wl_ref[...], preferred_element_type=jnp.float32)
         + bl_ref[...]
         + jnp.dot(h_ref[...], wr_ref[...], preferred_element_type=jnp.float32))
    h = jnp.maximum(h, 0.0)
    z = jnp.dot(h, wc_ref[...], preferred_element_type=jnp.float32) + bc_ref[...]
    m = jnp.max(z, axis=1, keepdims=True)
    e = jnp.exp(z - m)
    o_ref[...] = (z - m) - jnp.log(jnp.sum(e, axis=1, keepdims=True))


_acc_spec = pl.BlockSpec((2, ROWS_BLK, D), lambda i: (0, i, 0))
_deg_spec = pl.BlockSpec((NW, ROWS_BLK), lambda i: (0, i))
_row_spec = pl.BlockSpec((ROWS_BLK, D), lambda i: (i, 0))
_w_spec = pl.BlockSpec((D, D), lambda i: (0, 0))
_b_spec = pl.BlockSpec((1, D), lambda i: (0, 0))

_tc1 = pl.pallas_call(
    _tc1_body, grid=GRID,
    in_specs=[_acc_spec, _deg_spec, _row_spec, _w_spec, _b_spec, _w_spec],
    out_specs=_row_spec,
    out_shape=jax.ShapeDtypeStruct((N_NODES, D), jnp.float32))

_tc2 = pl.pallas_call(
    _tc2_body, grid=GRID,
    in_specs=[_acc_spec, _deg_spec, _row_spec, _w_spec, _b_spec, _w_spec,
              _w_spec, _b_spec],
    out_specs=_row_spec,
    out_shape=jax.ShapeDtypeStruct((N_NODES, D), jnp.float32))


@jax.jit
def kernel(x, edge_index, W1l, b1l, W1r, W2l, b2l, W2r, Wc, bc):
    ei = edge_index.astype(jnp.int32)
    src = ei[0]
    dst = ei[1]
    acc1, degp = _make_sc(True)(x, src, dst)
    degp = degp.reshape(NW, N_NODES)
    h1 = _tc1(acc1.reshape(2, N_NODES, D), degp, x,
              W1l.T, b1l.reshape(1, D), W1r.T)
    acc2, = _make_sc(False)(h1, src, dst)
    out = _tc2(acc2.reshape(2, N_NODES, D), degp, h1,
               W2l.T, b2l.reshape(1, D), W2r.T, Wc.T, bc.reshape(1, D))
    return out


# speedup vs baseline: 13.0700x; 2.2836x over previous
import functools

import jax
import jax.numpy as jnp
from jax import lax
from jax.experimental import pallas as pl
from jax.experimental.pallas import tpu as pltpu
from jax.experimental.pallas import tpu_sc as plsc

N_NODES = 10000
N_EDGES = 320000
D = 128

NC = 2
NS = 16
NW = NC * NS
EPT = N_EDGES // NW
CH = 80
NSTEP = EPT // CH
RPT = 624


def _sc_body(compute_deg, x_hbm, src_hbm, dst_hbm, *refs):
    if compute_deg:
        (acc_hbm, deg_hbm, acc_sh, rows0, rows1, sall, didx0, didx1,
         sem0, sem1, dsem0, dsem1, degp) = refs
    else:
        (acc_hbm, acc_sh, rows0, rows1, sall, didx0, didx1,
         sem0, sem1, dsem0, dsem1) = refs

    c = lax.axis_index("c")
    s = lax.axis_index("s")
    wid = s * NC + c
    ebase = wid * EPT
    rbase = s * RPT

    @pl.loop(0, CH)
    def _zero_rows(i):
        for j in range(D // 16):
            z = jnp.zeros((16,), jnp.float32)
            rows0[i, pl.ds(j * 16, 16)] = z
            rows1[i, pl.ds(j * 16, 16)] = z

    @pl.loop(0, 7)
    def _zero_acc(k):
        pltpu.sync_copy(rows0, acc_sh.at[pl.ds(rbase + k * CH, CH)])

    pltpu.sync_copy(rows1.at[pl.ds(0, RPT - 7 * CH)],
                    acc_sh.at[pl.ds(rbase + 7 * CH, RPT - 7 * CH)])

    @pl.when(s == NS - 1)
    def _zero_tail():
        pltpu.sync_copy(rows1.at[pl.ds(0, N_NODES - NS * RPT)],
                        acc_sh.at[pl.ds(NS * RPT, N_NODES - NS * RPT)])

    if compute_deg:
        @pl.loop(0, N_NODES // 16)
        def _zero_deg(i):
            degp[pl.ds(i * 16, 16)] = jnp.zeros((16,), jnp.float32)

        pltpu.sync_copy(dst_hbm.at[pl.ds(ebase, EPT)], sall)
        ones = jnp.ones((16,), jnp.float32)

        @pl.loop(0, EPT // 16)
        def _deg(i):
            idx = sall[pl.ds(i * 16, 16)]
            plsc.addupdate_scatter(degp, [idx], ones)

        pltpu.sync_copy(degp, deg_hbm.at[pl.ds(wid * N_NODES, N_NODES)])

    pltpu.sync_copy(src_hbm.at[pl.ds(ebase, EPT)], sall)

    plsc.subcore_barrier()

    def _gather(i, buf, sem):
        pltpu.async_copy(x_hbm.at[sall.at[pl.ds(i * CH, CH)]], buf, sem)

    def _gwait(buf, sem):
        pltpu.make_async_copy(x_hbm.at[pl.ds(0, CH)], buf, sem).wait()

    def _didx(i, buf, sem):
        pltpu.async_copy(dst_hbm.at[pl.ds(ebase + i * CH, CH)], buf, sem)

    def _dwait(buf, sem):
        pltpu.make_async_copy(dst_hbm.at[pl.ds(0, CH)], buf, sem).wait()

    def _scat(buf, dbuf):
        pltpu.sync_copy(buf, acc_sh.at[dbuf], add=True)

    _didx(0, didx0, dsem0)
    _gather(0, rows0, sem0)

    @pl.loop(0, (NSTEP - 1) // 2)
    def _edges(j):
        i = 2 * j
        _gather(i + 1, rows1, sem1)
        _didx(i + 1, didx1, dsem1)
        _gwait(rows0, sem0)
        _dwait(didx0, dsem0)
        _scat(rows0, didx0)
        _gather(i + 2, rows0, sem0)
        _didx(i + 2, didx0, dsem0)
        _gwait(rows1, sem1)
        _dwait(didx1, dsem1)
        _scat(rows1, didx1)

    _gwait(rows0, sem0)
    _dwait(didx0, dsem0)
    _scat(rows0, didx0)

    plsc.subcore_barrier()

    pltpu.sync_copy(acc_sh.at[pl.ds(rbase, RPT)],
                    acc_hbm.at[pl.ds(c * N_NODES + rbase, RPT)])

    @pl.when(s == NS - 1)
    def _write_tail():
        pltpu.sync_copy(acc_sh.at[pl.ds(NS * RPT, N_NODES - NS * RPT)],
                        acc_hbm.at[pl.ds(c * N_NODES + NS * RPT,
                                         N_NODES - NS * RPT)])


@functools.lru_cache(maxsize=None)
def _make_sc(compute_deg):
    mesh = plsc.VectorSubcoreMesh(core_axis_name="c", subcore_axis_name="s",
                                  num_cores=NC, num_subcores=NS)
    out_type = [jax.ShapeDtypeStruct((NC * N_NODES, D), jnp.float32)]
    if compute_deg:
        out_type.append(jax.ShapeDtypeStruct((NW * N_NODES,), jnp.float32))
    scratch = [
        pltpu.VMEM_SHARED((N_NODES, D), jnp.float32),
        pltpu.VMEM((CH, D), jnp.float32),
        pltpu.VMEM((CH, D), jnp.float32),
        pltpu.VMEM((EPT,), jnp.int32),
        pltpu.VMEM((CH,), jnp.int32),
        pltpu.VMEM((CH,), jnp.int32),
        pltpu.SemaphoreType.DMA,
        pltpu.SemaphoreType.DMA,
        pltpu.SemaphoreType.DMA,
        pltpu.SemaphoreType.DMA,
    ]
    if compute_deg:
        scratch += [
            pltpu.VMEM((N_NODES,), jnp.float32),
        ]
    return pl.kernel(functools.partial(_sc_body, compute_deg),
                     out_type=tuple(out_type), mesh=mesh,
                     scratch_types=tuple(scratch),
                     compiler_params=pltpu.CompilerParams(
                         needs_layout_passes=False))


ROWS_BLK = N_NODES
GRID = (1,)


def _mean(acc_ref, deg_ref):
    deg = jnp.sum(deg_ref[...], axis=0)
    inv = 1.0 / jnp.maximum(deg, 1.0)
    return (acc_ref[0] + acc_ref[1]) * inv[:, None]


def _tc1_body(acc_ref, deg_ref, x_ref, wl_ref, bl_ref, wr_ref, o_ref):
    mean = _mean(acc_ref, deg_ref)
    h = (jnp.dot(mean, wl_ref[...], preferred_element_type=jnp.float32)
         + bl_ref[...]
         + jnp.dot(x_ref[...], wr_ref[...], preferred_element_type=jnp.float32))
    o_ref[...] = jnp.maximum(h, 0.0)


def _tc2_body(acc_ref, deg_ref, h_ref, wl_ref, bl_ref, wr_ref, wc_ref, bc_ref,
              o_ref):
    mean = _mean(acc_ref, deg_ref)
    h = (jnp.dot(mean, wl_ref[...], preferred_element_type=jnp.float32)
         + bl_ref[...]
         + jnp.dot(h_ref[...], wr_ref[...], preferred_element_type=jnp.float32))
    h = jnp.maximum(h, 0.0)
    z = jnp.dot(h, wc_ref[...], preferred_element_type=jnp.float32) + bc_ref[...]
    m = jnp.max(z, axis=1, keepdims=True)
    e = jnp.exp(z - m)
    o_ref[...] = (z - m) - jnp.log(jnp.sum(e, axis=1, keepdims=True))


_acc_spec = pl.BlockSpec((2, ROWS_BLK, D), lambda i: (0, i, 0))
_deg_spec = pl.BlockSpec((NW, ROWS_BLK), lambda i: (0, i))
_row_spec = pl.BlockSpec((ROWS_BLK, D), lambda i: (i, 0))
_w_spec = pl.BlockSpec((D, D), lambda i: (0, 0))
_b_spec = pl.BlockSpec((1, D), lambda i: (0, 0))

_tc1 = pl.pallas_call(
    _tc1_body, grid=GRID,
    in_specs=[_acc_spec, _deg_spec, _row_spec, _w_spec, _b_spec, _w_spec],
    out_specs=_row_spec,
    out_shape=jax.ShapeDtypeStruct((N_NODES, D), jnp.float32))

_tc2 = pl.pallas_call(
    _tc2_body, grid=GRID,
    in_specs=[_acc_spec, _deg_spec, _row_spec, _w_spec, _b_spec, _w_spec,
              _w_spec, _b_spec],
    out_specs=_row_spec,
    out_shape=jax.ShapeDtypeStruct((N_NODES, D), jnp.float32))


@jax.jit
def kernel(x, edge_index, W1l, b1l, W1r, W2l, b2l, W2r, Wc, bc):
    ei = edge_index.astype(jnp.int32)
    src = ei[0]
    dst = ei[1]
    acc1, degp = _make_sc(True)(x, src, dst)
    degp = degp.reshape(NW, N_NODES)
    h1 = _tc1(acc1.reshape(2, N_NODES, D), degp, x,
              W1l.T, b1l.reshape(1, D), W1r.T)
    acc2, = _make_sc(False)(h1, src, dst)
    out = _tc2(acc2.reshape(2, N_NODES, D), degp, h1,
               W2l.T, b2l.reshape(1, D), W2r.T, Wc.T, bc.reshape(1, D))
    return out


# speedup vs baseline: 15.5441x; 1.1893x over previous
import functools

import jax
import jax.numpy as jnp
from jax import lax
from jax.experimental import pallas as pl
from jax.experimental.pallas import tpu as pltpu
from jax.experimental.pallas import tpu_sc as plsc

N_NODES = 10000
N_EDGES = 320000
D = 128

NC = 2
NS = 16
NW = NC * NS
EPT = N_EDGES // NW
CH = 80
NSTEP = EPT // CH
RPT = 624
DEPTH = 3


def _sc_body(compute_deg, x_hbm, src_hbm, dst_hbm, *refs):
    if compute_deg:
        acc_hbm, deg_hbm = refs[0], refs[1]
        rest = refs[2:]
    else:
        acc_hbm = refs[0]
        rest = refs[1:]
    acc_sh = rest[0]
    rows = rest[1:1 + DEPTH]
    sall = rest[1 + DEPTH]
    didx = rest[2 + DEPTH:2 + 2 * DEPTH]
    gsem = rest[2 + 2 * DEPTH:2 + 3 * DEPTH]
    dsem = rest[2 + 3 * DEPTH:2 + 4 * DEPTH]

    c = lax.axis_index("c")
    s = lax.axis_index("s")
    wid = s * NC + c
    ebase = wid * EPT
    rbase = s * RPT

    def _zero_buf(buf):
        @pl.loop(0, CH)
        def _z(i):
            for j in range(D // 16):
                buf[i, pl.ds(j * 16, 16)] = jnp.zeros((16,), jnp.float32)

    if compute_deg:
        _zero_buf(rows[0])
        pltpu.sync_copy(dst_hbm.at[pl.ds(ebase, EPT)], sall)
        ones = jnp.ones((16,), jnp.float32)

        @pl.loop(0, EPT // 16)
        def _deg(i):
            idx = sall[pl.ds(i * 16, 16)]
            plsc.addupdate_scatter(
                rows[0],
                [lax.shift_right_logical(idx, 7), lax.bitwise_and(idx, 127)],
                ones)

        pltpu.sync_copy(rows[0], deg_hbm.at[wid])

    for b in range(DEPTH):
        _zero_buf(rows[b])

    @pl.loop(0, 7)
    def _zero_acc(k):
        pltpu.sync_copy(rows[0], acc_sh.at[pl.ds(rbase + k * CH, CH)])

    pltpu.sync_copy(rows[1].at[pl.ds(0, RPT - 7 * CH)],
                    acc_sh.at[pl.ds(rbase + 7 * CH, RPT - 7 * CH)])

    @pl.when(s == NS - 1)
    def _zero_tail():
        pltpu.sync_copy(rows[1].at[pl.ds(0, N_NODES - NS * RPT)],
                        acc_sh.at[pl.ds(NS * RPT, N_NODES - NS * RPT)])

    pltpu.sync_copy(src_hbm.at[pl.ds(ebase, EPT)], sall)

    plsc.subcore_barrier()

    def _start(i, t):
        pltpu.async_copy(dst_hbm.at[pl.ds(ebase + i * CH, CH)],
                         didx[t], dsem[t])
        pltpu.async_copy(x_hbm.at[sall.at[pl.ds(i * CH, CH)]],
                         rows[t], gsem[t])

    def _finish(t):
        pltpu.make_async_copy(dst_hbm.at[pl.ds(0, CH)], didx[t],
                              dsem[t]).wait()
        pltpu.make_async_copy(x_hbm.at[pl.ds(0, CH)], rows[t],
                              gsem[t]).wait()
        pltpu.sync_copy(rows[t], acc_sh.at[didx[t]], add=True)

    for t in range(DEPTH):
        _start(t, t)

    NFULL = (NSTEP - DEPTH) // DEPTH

    @pl.loop(0, NFULL)
    def _edges(j):
        for t in range(DEPTH):
            i = DEPTH * j + t
            _finish(t)
            _start(i + DEPTH, t)

    for i in range(DEPTH * NFULL, NSTEP):
        t = i % DEPTH
        _finish(t)
        if i + DEPTH < NSTEP:
            _start(i + DEPTH, t)

    plsc.subcore_barrier()

    pltpu.sync_copy(acc_sh.at[pl.ds(rbase, RPT)],
                    acc_hbm.at[pl.ds(c * N_NODES + rbase, RPT)])

    @pl.when(s == NS - 1)
    def _write_tail():
        pltpu.sync_copy(acc_sh.at[pl.ds(NS * RPT, N_NODES - NS * RPT)],
                        acc_hbm.at[pl.ds(c * N_NODES + NS * RPT,
                                         N_NODES - NS * RPT)])


@functools.lru_cache(maxsize=None)
def _make_sc(compute_deg):
    mesh = plsc.VectorSubcoreMesh(core_axis_name="c", subcore_axis_name="s",
                                  num_cores=NC, num_subcores=NS)
    out_type = [jax.ShapeDtypeStruct((NC * N_NODES, D), jnp.float32)]
    if compute_deg:
        out_type.append(jax.ShapeDtypeStruct((NW, CH, D), jnp.float32))
    scratch = (
        [pltpu.VMEM_SHARED((N_NODES, D), jnp.float32)]
        + [pltpu.VMEM((CH, D), jnp.float32)] * DEPTH
        + [pltpu.VMEM((EPT,), jnp.int32)]
        + [pltpu.VMEM((CH,), jnp.int32)] * DEPTH
        + [pltpu.SemaphoreType.DMA] * (2 * DEPTH)
    )
    return pl.kernel(functools.partial(_sc_body, compute_deg),
                     out_type=tuple(out_type), mesh=mesh,
                     scratch_types=tuple(scratch),
                     compiler_params=pltpu.CompilerParams(
                         needs_layout_passes=False))


ROWS_BLK = N_NODES
GRID = (1,)


def _mean(acc_ref, deg_ref):
    deg = jnp.sum(deg_ref[...], axis=0)[:N_NODES]
    inv = 1.0 / jnp.maximum(deg, 1.0)
    return (acc_ref[0] + acc_ref[1]) * inv[:, None]


def _tc1_body(acc_ref, deg_ref, x_ref, wl_ref, bl_ref, wr_ref, o_ref):
    mean = _mean(acc_ref, deg_ref)
    h = (jnp.dot(mean, wl_ref[...], preferred_element_type=jnp.float32)
         + bl_ref[...]
         + jnp.dot(x_ref[...], wr_ref[...], preferred_element_type=jnp.float32))
    o_ref[...] = jnp.maximum(h, 0.0)


def _tc2_body(acc_ref, deg_ref, h_ref, wl_ref, bl_ref, wr_ref, wc_ref, bc_ref,
              o_ref):
    mean = _mean(acc_ref, deg_ref)
    h = (jnp.dot(mean, wl_ref[...], preferred_element_type=jnp.float32)
         + bl_ref[...]
         + jnp.dot(h_ref[...], wr_ref[...], preferred_element_type=jnp.float32))
    h = jnp.maximum(h, 0.0)
    z = jnp.dot(h, wc_ref[...], preferred_element_type=jnp.float32) + bc_ref[...]
    m = jnp.max(z, axis=1, keepdims=True)
    e = jnp.exp(z - m)
    o_ref[...] = (z - m) - jnp.log(jnp.sum(e, axis=1, keepdims=True))


_acc_spec = pl.BlockSpec((2, ROWS_BLK, D), lambda i: (0, i, 0))
_deg_spec = pl.BlockSpec((NW, CH * D), lambda i: (0, 0))
_row_spec = pl.BlockSpec((ROWS_BLK, D), lambda i: (i, 0))
_w_spec = pl.BlockSpec((D, D), lambda i: (0, 0))
_b_spec = pl.BlockSpec((1, D), lambda i: (0, 0))

_tc1 = pl.pallas_call(
    _tc1_body, grid=GRID,
    in_specs=[_acc_spec, _deg_spec, _row_spec, _w_spec, _b_spec, _w_spec],
    out_specs=_row_spec,
    out_shape=jax.ShapeDtypeStruct((N_NODES, D), jnp.float32))

_tc2 = pl.pallas_call(
    _tc2_body, grid=GRID,
    in_specs=[_acc_spec, _deg_spec, _row_spec, _w_spec, _b_spec, _w_spec,
              _w_spec, _b_spec],
    out_specs=_row_spec,
    out_shape=jax.ShapeDtypeStruct((N_NODES, D), jnp.float32))


@jax.jit
def kernel(x, edge_index, W1l, b1l, W1r, W2l, b2l, W2r, Wc, bc):
    ei = edge_index.astype(jnp.int32)
    src = ei[0]
    dst = ei[1]
    acc1, degp = _make_sc(True)(x, src, dst)
    degp = degp.reshape(NW, CH * D)
    h1 = _tc1(acc1.reshape(2, N_NODES, D), degp, x,
              W1l.T, b1l.reshape(1, D), W1r.T)
    acc2, = _make_sc(False)(h1, src, dst)
    out = _tc2(acc2.reshape(2, N_NODES, D), degp, h1,
               W2l.T, b2l.reshape(1, D), W2r.T, Wc.T, bc.reshape(1, D))
    return out
